# baseline (speedup 1.0000x reference)
import functools

import jax
import jax.numpy as jnp
from jax import lax
from jax.experimental import pallas as pl
from jax.experimental.pallas import tpu as pltpu
from jax.experimental.pallas import tpu_sc as plsc

DIM = 64
B = 16384
NC, NS, L = 2, 16, 16
NW = NC * NS
PER_W = B // NW
CHUNK = 128
NCHUNK = PER_W // CHUNK
GROUPS = CHUNK // L


def _build():
    mesh = plsc.VectorSubcoreMesh(core_axis_name="c", subcore_axis_name="s")
    out_t = (jax.ShapeDtypeStruct((B,), jnp.float32),
             jax.ShapeDtypeStruct((B,), jnp.float32))
    scratch = [
        pltpu.VMEM((PER_W,), jnp.int32),
        pltpu.VMEM((PER_W,), jnp.int32),
        pltpu.VMEM((PER_W,), jnp.int32),
        pltpu.VMEM((CHUNK, DIM), jnp.float32),
        pltpu.VMEM((CHUNK, DIM), jnp.float32),
        pltpu.VMEM((CHUNK, DIM), jnp.float32),
        pltpu.VMEM((PER_W,), jnp.float32),
        pltpu.SemaphoreType.DMA,
        pltpu.SemaphoreType.DMA,
        pltpu.SemaphoreType.DMA,
    ]

    @functools.partial(
        pl.kernel, out_type=out_t, mesh=mesh, scratch_types=scratch,
        compiler_params=pltpu.CompilerParams(needs_layout_passes=False,
                                             use_tc_tiling_on_sc=False))
    def trans_e(pos_h, pos_r, pos_t, neg_h, neg_r, neg_t, ent, rel,
                pos_out, neg_out,
                hidx, ridx, tidx, hrows, rrows, trows, scores,
                sem_h, sem_r, sem_t):
        wid = lax.axis_index("s") * NC + lax.axis_index("c")
        base = wid * PER_W
        for h_in, r_in, t_in, out in ((pos_h, pos_r, pos_t, pos_out),
                                      (neg_h, neg_r, neg_t, neg_out)):
            pltpu.sync_copy(h_in.at[pl.ds(base, PER_W)], hidx)
            pltpu.sync_copy(r_in.at[pl.ds(base, PER_W)], ridx)
            pltpu.sync_copy(t_in.at[pl.ds(base, PER_W)], tidx)

            @pl.loop(0, NCHUNK)
            def _chunk(c):
                off = c * CHUNK
                cp_h = pltpu.async_copy(
                    ent.at[hidx.at[pl.ds(off, CHUNK)]], hrows, sem_h)
                cp_r = pltpu.async_copy(
                    rel.at[ridx.at[pl.ds(off, CHUNK)]], rrows, sem_r)
                cp_t = pltpu.async_copy(
                    ent.at[tidx.at[pl.ds(off, CHUNK)]], trows, sem_t)
                cp_h.wait()
                cp_r.wait()
                cp_t.wait()

                @pl.loop(0, GROUPS)
                def _group(g):
                    lanes = lax.iota(jnp.int32, L)
                    acc = jnp.zeros((L,), jnp.float32)
                    for i in range(L):
                        row = g * L + i
                        s = jnp.zeros((L,), jnp.float32)
                        for k in range(DIM // L):
                            hv = hrows[row, pl.ds(k * L, L)]
                            rv = rrows[row, pl.ds(k * L, L)]
                            tv = trows[row, pl.ds(k * L, L)]
                            s = s + jnp.abs(hv + rv - tv)
                        tot = jnp.sum(s)
                        acc = jnp.where(lanes == i, tot, acc)
                    scores[pl.ds(off + g * L, L)] = acc

            pltpu.sync_copy(scores, out.at[pl.ds(base, PER_W)])

    return trans_e


_trans_e = _build()


def kernel(pos_h, pos_r, pos_t, neg_h, neg_r, neg_t, entity_emb, relation_emb):
    return _trans_e(pos_h, pos_r, pos_t, neg_h, neg_r, neg_t,
                    entity_emb, relation_emb)
